# baseline (speedup 1.0000x reference)
import jax
import jax.numpy as jnp
from jax.experimental import pallas as pl
from jax.experimental.pallas import tpu as pltpu


def _copy_body(nt_ref, k_ref, kc_ref, v_ref, vc_ref, ok_ref, ov_ref,
               sem_k, sem_v, sem_rk, sem_rv):
    bulk_k = pltpu.make_async_copy(kc_ref, ok_ref, sem_k)
    bulk_v = pltpu.make_async_copy(vc_ref, ov_ref, sem_v)
    bulk_k.start()
    bulk_v.start()
    bulk_k.wait()
    bulk_v.wait()
    nt = nt_ref[0]
    row_k = pltpu.make_async_copy(k_ref, ok_ref.at[:, pl.ds(nt, 1), :], sem_rk)
    row_v = pltpu.make_async_copy(v_ref, ov_ref.at[:, pl.ds(nt, 1), :], sem_rv)
    row_k.start()
    row_v.start()
    row_k.wait()
    row_v.wait()


def kernel(k, k_cache, v, v_cache, n_tokens):
    B, H, S, D = k_cache.shape
    BH = B * H
    nt = jnp.asarray(n_tokens, jnp.int32).reshape(1)
    k2 = k.reshape(BH, 1, D)
    v2 = v.reshape(BH, 1, D)
    kc = k_cache.reshape(BH, S, D)
    vc = v_cache.reshape(BH, S, D)

    any_spec = pl.BlockSpec(memory_space=pl.ANY)
    out_k, out_v = pl.pallas_call(
        _copy_body,
        in_specs=[
            pl.BlockSpec(memory_space=pltpu.SMEM),
            any_spec, any_spec, any_spec, any_spec,
        ],
        out_specs=[any_spec, any_spec],
        out_shape=[
            jax.ShapeDtypeStruct((BH, S, D), k_cache.dtype),
            jax.ShapeDtypeStruct((BH, S, D), v_cache.dtype),
        ],
        scratch_shapes=[pltpu.SemaphoreType.DMA] * 4,
    )(nt, k2, kc, v2, vc)
    return (out_k.reshape(B, H, S, D), out_v.reshape(B, H, S, D))


# speedup vs baseline: 30.6562x; 30.6562x over previous
import jax
import jax.numpy as jnp
from jax.experimental import pallas as pl
from jax.experimental.pallas import tpu as pltpu


def _body(nt_ref, k_ref, kc_ref, v_ref, vc_ref, ok_ref, ov_ref):
    nt = nt_ref[0]
    s0 = pl.program_id(1) * kc_ref.shape[0]
    rows = jax.lax.broadcasted_iota(jnp.int32, (kc_ref.shape[0], 1), 0) + s0
    mask = rows == nt
    ok_ref[...] = jnp.where(mask, k_ref[...], kc_ref[...])
    ov_ref[...] = jnp.where(mask, v_ref[...], vc_ref[...])


def kernel(k, k_cache, v, v_cache, n_tokens):
    B, H, S, D = k_cache.shape
    BH = B * H
    SBLK = 1024
    nt = jnp.asarray(n_tokens, jnp.int32).reshape(1)
    k2 = k.reshape(BH, 1, D)
    v2 = v.reshape(BH, 1, D)
    kc = k_cache.reshape(BH, S, D)
    vc = v_cache.reshape(BH, S, D)

    cache_spec = pl.BlockSpec((None, SBLK, D), lambda i, j: (i, j, 0))
    row_spec = pl.BlockSpec((None, 1, D), lambda i, j: (i, 0, 0))
    out_k, out_v = pl.pallas_call(
        _body,
        grid=(BH, S // SBLK),
        in_specs=[
            pl.BlockSpec(memory_space=pltpu.SMEM),
            row_spec, cache_spec, row_spec, cache_spec,
        ],
        out_specs=[cache_spec, cache_spec],
        out_shape=[
            jax.ShapeDtypeStruct((BH, S, D), k_cache.dtype),
            jax.ShapeDtypeStruct((BH, S, D), v_cache.dtype),
        ],
        compiler_params=pltpu.CompilerParams(
            dimension_semantics=("parallel", "parallel"),
        ),
    )(nt, k2, kc, v2, vc)
    return (out_k.reshape(B, H, S, D), out_v.reshape(B, H, S, D))
